# baseline (speedup 1.0000x reference)
import functools
import math

import jax
import jax.numpy as jnp
from jax import lax
from jax.experimental import pallas as pl
from jax.experimental.pallas import tpu as pltpu
from jax.experimental.pallas import tpu_sc as plsc

B, P, D, T, TOP_K = 4096, 64, 2048, 8, 2
KBLK = 128
BBLK = 1024

NC, NS, L = 2, 16, 16
NW = NC * NS
RPW = B // NW


def _sim_kernel(q_ref, k_ref, t_ref, simT_ref):
    q = q_ref[...]
    kk = k_ref[...]
    qn = q / jnp.maximum(jnp.sqrt(jnp.sum(q * q, axis=1, keepdims=True)), 1e-12)
    kn = kk / jnp.maximum(jnp.sqrt(jnp.sum(kk * kk, axis=1, keepdims=True)), 1e-12)
    temp = jnp.clip(t_ref[0, 0], 0.1, 2.0)
    simT_ref[...] = lax.dot_general(
        kn, qn, (((1,), (1,)), ((), ())),
        preferred_element_type=jnp.float32) / temp


def _pv_matmul_kernel(pv_ref, w_ref, out_ref):
    k = pl.program_id(0)

    @pl.when(k == 0)
    def _init():
        out_ref[...] = jnp.zeros_like(out_ref)

    for t in range(T):
        out_ref[...] += jnp.dot(pv_ref[:, t, :], w_ref[t, :, :],
                                preferred_element_type=jnp.float32)


def _sc_route_body(simT_hbm, idxT_hbm, aT_hbm, simT_v, idx0_v, idx1_v, aT_v):
    wid = lax.axis_index("s") * NC + lax.axis_index("c")
    base = wid * RPW
    pltpu.sync_copy(simT_hbm.at[:, pl.ds(base, RPW)], simT_v)

    def chunk_body(ch, carry):
        off = ch * L

        def scan_col(c, st):
            m0, m1, i0, i1 = st
            v = simT_v[c, pl.ds(off, L)]
            cv = jnp.full((L,), 1, jnp.int32) * c
            gt0 = v > m0
            gt1 = v > m1
            i1 = jnp.where(gt0, i0, jnp.where(gt1, cv, i1))
            m1 = jnp.where(gt0, m0, jnp.where(gt1, v, m1))
            i0 = jnp.where(gt0, cv, i0)
            m0 = jnp.where(gt0, v, m0)
            return (m0, m1, i0, i1)

        m0, m1, i0, i1 = lax.fori_loop(
            0, P, scan_col,
            (jnp.full((L,), -jnp.inf, jnp.float32),
             jnp.full((L,), -jnp.inf, jnp.float32),
             jnp.zeros((L,), jnp.int32),
             jnp.zeros((L,), jnp.int32)))

        t = jnp.exp(m1 - m0)
        w0 = 1.0 / (1.0 + t)
        w1 = t / (1.0 + t)

        def fill_col(c, carry2):
            cv = jnp.full((L,), 1, jnp.int32) * c
            a_c = (jnp.where(i0 == cv, w0, 0.0)
                   + jnp.where(i1 == cv, w1, 0.0))
            aT_v[c, pl.ds(off, L)] = a_c
            return carry2

        lax.fori_loop(0, P, fill_col, 0)

        idx0_v[pl.ds(off, L)] = i0
        idx1_v[pl.ds(off, L)] = i1
        return carry

    lax.fori_loop(0, RPW // L, chunk_body, 0)

    pltpu.sync_copy(idx0_v, idxT_hbm.at[0, pl.ds(base, RPW)])
    pltpu.sync_copy(idx1_v, idxT_hbm.at[1, pl.ds(base, RPW)])
    pltpu.sync_copy(aT_v, aT_hbm.at[:, pl.ds(base, RPW)])


_sc_route = functools.partial(
    pl.kernel,
    mesh=plsc.VectorSubcoreMesh(core_axis_name="c", subcore_axis_name="s"),
    out_type=[
        jax.ShapeDtypeStruct((TOP_K, B), jnp.int32),
        jax.ShapeDtypeStruct((P, B), jnp.float32),
    ],
    scratch_types=[
        pltpu.VMEM((P, RPW), jnp.float32),
        pltpu.VMEM((RPW,), jnp.int32),
        pltpu.VMEM((RPW,), jnp.int32),
        pltpu.VMEM((P, RPW), jnp.float32),
    ],
)(_sc_route_body)


def _agg_kernel(aT_ref, idxT_ref, pv_ref, b_ref, g_ref, be_ref,
                out_ref, idx_ref):
    h = lax.dot_general(
        aT_ref[...], pv_ref[...], (((0,), (0,)), ((), ())),
        preferred_element_type=jnp.float32) + b_ref[...]

    i0 = idxT_ref[...][0:1, :].reshape(idx_ref.shape[0], 1)
    i1 = idxT_ref[...][1:2, :].reshape(idx_ref.shape[0], 1)
    idx_ref[...] = jnp.concatenate([i0, i1], axis=1)

    n_inv = 1.0 / D
    mu = jnp.sum(h, axis=1, keepdims=True) * n_inv
    var = jnp.sum(h * h, axis=1, keepdims=True) * n_inv - mu * mu
    s = jax.lax.rsqrt(var + 1e-5)
    scale = s * g_ref[...]
    shift = be_ref[...] - mu * scale
    hn = h * scale + shift

    half = 0.5 * hn
    out_ref[...] = half + half * jax.lax.erf(hn * (1.0 / math.sqrt(2.0)))


@jax.jit
def kernel(query_feature, prompt_keys, prompt_values, temperature, W, b,
           gamma, beta):
    t2 = jnp.asarray(temperature, jnp.float32).reshape(1, 1)

    simT = pl.pallas_call(
        _sim_kernel,
        grid=(B // BBLK,),
        in_specs=[
            pl.BlockSpec((BBLK, D), lambda i: (i, 0)),
            pl.BlockSpec((P, D), lambda i: (0, 0)),
            pl.BlockSpec((1, 1), lambda i: (0, 0)),
        ],
        out_specs=pl.BlockSpec((P, BBLK), lambda i: (0, i)),
        out_shape=jax.ShapeDtypeStruct((P, B), jnp.float32),
    )(query_feature, prompt_keys, t2)

    w3 = W.reshape(T, D, D)
    pv_table = pl.pallas_call(
        _pv_matmul_kernel,
        grid=(D // KBLK,),
        in_specs=[
            pl.BlockSpec((P, T, KBLK), lambda k: (0, 0, k)),
            pl.BlockSpec((T, KBLK, D), lambda k: (0, k, 0)),
        ],
        out_specs=pl.BlockSpec((P, D), lambda k: (0, 0)),
        out_shape=jax.ShapeDtypeStruct((P, D), jnp.float32),
    )(prompt_values, w3)

    idx_t, a_t = _sc_route(simT)

    out, idx = pl.pallas_call(
        _agg_kernel,
        grid=(B // BBLK,),
        in_specs=[
            pl.BlockSpec((P, BBLK), lambda i: (0, i)),
            pl.BlockSpec((TOP_K, BBLK), lambda i: (0, i)),
            pl.BlockSpec((P, D), lambda i: (0, 0)),
            pl.BlockSpec((1, D), lambda i: (0, 0)),
            pl.BlockSpec((1, D), lambda i: (0, 0)),
            pl.BlockSpec((1, D), lambda i: (0, 0)),
        ],
        out_specs=[
            pl.BlockSpec((BBLK, D), lambda i: (i, 0)),
            pl.BlockSpec((BBLK, TOP_K), lambda i: (i, 0)),
        ],
        out_shape=[
            jax.ShapeDtypeStruct((B, D), jnp.float32),
            jax.ShapeDtypeStruct((B, TOP_K), jnp.int32),
        ],
    )(a_t, idx_t, pv_table, b.reshape(1, D), gamma.reshape(1, D),
      beta.reshape(1, D))

    return (out, idx)


# speedup vs baseline: 1.0284x; 1.0284x over previous
import functools
import math

import jax
import jax.numpy as jnp
from jax import lax
from jax.experimental import pallas as pl
from jax.experimental.pallas import tpu as pltpu
from jax.experimental.pallas import tpu_sc as plsc

B, P, D, T, TOP_K = 4096, 64, 2048, 8, 2
KBLK = 128
BBLK = 1024

NC, NS, L = 2, 16, 16
NW = NC * NS
RPW = B // NW


def _sim_kernel(q_ref, k_ref, t_ref, simT_ref):
    q = q_ref[...]
    kk = k_ref[...]
    qn = q / jnp.maximum(jnp.sqrt(jnp.sum(q * q, axis=1, keepdims=True)), 1e-12)
    kn = kk / jnp.maximum(jnp.sqrt(jnp.sum(kk * kk, axis=1, keepdims=True)), 1e-12)
    temp = jnp.clip(t_ref[0, 0], 0.1, 2.0)
    simT_ref[...] = lax.dot_general(
        kn, qn, (((1,), (1,)), ((), ())),
        preferred_element_type=jnp.float32) / temp


def _pv_matmul_kernel(pv_ref, w_ref, out_ref):
    k = pl.program_id(0)

    @pl.when(k == 0)
    def _init():
        out_ref[...] = jnp.zeros_like(out_ref)

    for t in range(T):
        out_ref[...] += jnp.dot(pv_ref[:, t, :], w_ref[t, :, :],
                                preferred_element_type=jnp.float32)


def _sc_route_body(simT_hbm, idxT_hbm, aT_hbm, simT_v, idx0_v, idx1_v, aT_v):
    wid = lax.axis_index("s") * NC + lax.axis_index("c")
    base = wid * RPW
    pltpu.sync_copy(simT_hbm.at[:, pl.ds(base, RPW)], simT_v)

    def chunk_body(ch, carry):
        off = ch * L

        def scan_col(c, st):
            m0, m1, i0, i1 = st
            v = simT_v[c, pl.ds(off, L)]
            cv = jnp.full((L,), 1, jnp.int32) * c
            gt0 = v > m0
            gt1 = v > m1
            i1 = jnp.where(gt0, i0, jnp.where(gt1, cv, i1))
            m1 = jnp.where(gt0, m0, jnp.where(gt1, v, m1))
            i0 = jnp.where(gt0, cv, i0)
            m0 = jnp.where(gt0, v, m0)
            return (m0, m1, i0, i1)

        m0, m1, i0, i1 = lax.fori_loop(
            0, P, scan_col,
            (jnp.full((L,), -jnp.inf, jnp.float32),
             jnp.full((L,), -jnp.inf, jnp.float32),
             jnp.zeros((L,), jnp.int32),
             jnp.zeros((L,), jnp.int32)))

        t = jnp.exp(m1 - m0)
        w0 = 1.0 / (1.0 + t)
        w1 = t / (1.0 + t)

        def fill_col(c, carry2):
            cv = jnp.full((L,), 1, jnp.int32) * c
            a_c = (jnp.where(i0 == cv, w0, 0.0)
                   + jnp.where(i1 == cv, w1, 0.0))
            aT_v[c, pl.ds(off, L)] = a_c
            return carry2

        lax.fori_loop(0, P, fill_col, 0)

        idx0_v[pl.ds(off, L)] = i0
        idx1_v[pl.ds(off, L)] = i1
        return carry

    lax.fori_loop(0, RPW // L, chunk_body, 0)

    pltpu.sync_copy(idx0_v, idxT_hbm.at[0, pl.ds(base, RPW)])
    pltpu.sync_copy(idx1_v, idxT_hbm.at[1, pl.ds(base, RPW)])
    pltpu.sync_copy(aT_v, aT_hbm.at[:, pl.ds(base, RPW)])


_sc_route = functools.partial(
    pl.kernel,
    mesh=plsc.VectorSubcoreMesh(core_axis_name="c", subcore_axis_name="s"),
    out_type=[
        jax.ShapeDtypeStruct((TOP_K, B), jnp.int32),
        jax.ShapeDtypeStruct((P, B), jnp.float32),
    ],
    scratch_types=[
        pltpu.VMEM((P, RPW), jnp.float32),
        pltpu.VMEM((RPW,), jnp.int32),
        pltpu.VMEM((RPW,), jnp.int32),
        pltpu.VMEM((P, RPW), jnp.float32),
    ],
)(_sc_route_body)


def _agg_kernel(aT_ref, pv_ref, b_ref, g_ref, be_ref, out_ref):
    h = lax.dot_general(
        aT_ref[...], pv_ref[...], (((0,), (0,)), ((), ())),
        preferred_element_type=jnp.float32) + b_ref[...]

    n_inv = 1.0 / D
    mu = jnp.sum(h, axis=1, keepdims=True) * n_inv
    var = jnp.sum(h * h, axis=1, keepdims=True) * n_inv - mu * mu
    s = jax.lax.rsqrt(var + 1e-5)
    scale = s * g_ref[...]
    shift = be_ref[...] - mu * scale
    hn = h * scale + shift

    half = 0.5 * hn
    out_ref[...] = half + half * jax.lax.erf(hn * (1.0 / math.sqrt(2.0)))


@jax.jit
def kernel(query_feature, prompt_keys, prompt_values, temperature, W, b,
           gamma, beta):
    t2 = jnp.asarray(temperature, jnp.float32).reshape(1, 1)

    simT = pl.pallas_call(
        _sim_kernel,
        grid=(B // BBLK,),
        in_specs=[
            pl.BlockSpec((BBLK, D), lambda i: (i, 0)),
            pl.BlockSpec((P, D), lambda i: (0, 0)),
            pl.BlockSpec((1, 1), lambda i: (0, 0)),
        ],
        out_specs=pl.BlockSpec((P, BBLK), lambda i: (0, i)),
        out_shape=jax.ShapeDtypeStruct((P, B), jnp.float32),
    )(query_feature, prompt_keys, t2)

    w3 = W.reshape(T, D, D)
    pv_table = pl.pallas_call(
        _pv_matmul_kernel,
        grid=(D // KBLK,),
        in_specs=[
            pl.BlockSpec((P, T, KBLK), lambda k: (0, 0, k)),
            pl.BlockSpec((T, KBLK, D), lambda k: (0, k, 0)),
        ],
        out_specs=pl.BlockSpec((P, D), lambda k: (0, 0)),
        out_shape=jax.ShapeDtypeStruct((P, D), jnp.float32),
    )(prompt_values, w3)

    idx_t, a_t = _sc_route(simT)

    out = pl.pallas_call(
        _agg_kernel,
        grid=(B // BBLK,),
        in_specs=[
            pl.BlockSpec((P, BBLK), lambda i: (0, i)),
            pl.BlockSpec((P, D), lambda i: (0, 0)),
            pl.BlockSpec((1, D), lambda i: (0, 0)),
            pl.BlockSpec((1, D), lambda i: (0, 0)),
            pl.BlockSpec((1, D), lambda i: (0, 0)),
        ],
        out_specs=pl.BlockSpec((BBLK, D), lambda i: (i, 0)),
        out_shape=jax.ShapeDtypeStruct((B, D), jnp.float32),
    )(a_t, pv_table, b.reshape(1, D), gamma.reshape(1, D), beta.reshape(1, D))

    return (out, idx_t.T)
